# Initial kernel scaffold; baseline (speedup 1.0000x reference)
import functools

import jax
import jax.numpy as jnp
from jax import lax
from jax.experimental import pallas as pl
from jax.experimental.pallas import tpu as pltpu
from jax.experimental.pallas import tpu_sc as plsc

_B, _N, _E, _D = 8, 2048, 65536, 128
_LANES = 16
_NSLICE = _D // _LANES
_NWORK = 32
_CHUNK = 2048


def _spmm_sc(x, src, dst, val):
    mesh = plsc.VectorSubcoreMesh(core_axis_name="c", subcore_axis_name="s")

    @functools.partial(
        pl.kernel,
        out_type=jax.ShapeDtypeStruct((_B, _N, _D), jnp.float32),
        mesh=mesh,
        scratch_types=[
            pltpu.VMEM((_N, _LANES), jnp.float32),
            pltpu.VMEM((_N, _LANES), jnp.float32),
            pltpu.VMEM((_CHUNK,), jnp.int32),
            pltpu.VMEM((_CHUNK,), jnp.int32),
            pltpu.VMEM((_CHUNK,), jnp.float32),
        ],
    )
    def spmm_kernel(x_hbm, src_hbm, dst_hbm, val_hbm, a_hbm,
                    xv, acc, sb, db, vb):
        w = lax.axis_index("s") * 2 + lax.axis_index("c")

        @pl.loop(0, 2)
        def _item(k):
            itm = w + _NWORK * k
            b = itm // _NSLICE
            ds = itm % _NSLICE

            pltpu.sync_copy(x_hbm.at[b, :, pl.ds(ds * _LANES, _LANES)], xv)

            @pl.loop(0, _N)
            def _zero(i):
                acc[i, :] = jnp.zeros((_LANES,), jnp.float32)

            @pl.loop(0, _E // _CHUNK)
            def _chunk(ck):
                base = ck * _CHUNK
                pltpu.sync_copy(src_hbm.at[b, pl.ds(base, _CHUNK)], sb)
                pltpu.sync_copy(dst_hbm.at[b, pl.ds(base, _CHUNK)], db)
                pltpu.sync_copy(val_hbm.at[b, pl.ds(base, _CHUNK)], vb)

                @pl.loop(0, _CHUNK, step=_LANES)
                def _group(g):
                    sv = sb[pl.ds(g, _LANES)]
                    dv = db[pl.ds(g, _LANES)]
                    vv = vb[pl.ds(g, _LANES)]
                    for r in range(_LANES):
                        row = jnp.full((_LANES,), r, jnp.int32)
                        xr = plsc.load_gather(xv, [sv, row])
                        plsc.addupdate_scatter(acc, [dv, row], xr * vv)

            pltpu.sync_copy(acc, a_hbm.at[b, :, pl.ds(ds * _LANES, _LANES)])

    return spmm_kernel(x, src, dst, val)


_BN = 256


def _gru_body(a_ref, x_ref, m_ref, wz0, wz1, wr0, wr1, wh0, wh1,
              bz, br, bh, o_ref):
    a = a_ref[0]
    xb = x_ref[0]
    m = m_ref[0]
    dot = functools.partial(jnp.dot, preferred_element_type=jnp.float32)
    z = jax.nn.sigmoid(dot(a, wz0[...]) + dot(xb, wz1[...]) + bz[...])
    r = jax.nn.sigmoid(dot(a, wr0[...]) + dot(xb, wr1[...]) + br[...])
    h = jnp.tanh(m * (dot(a, wh0[...]) + dot(r * xb, wh1[...]) + bh[...]))
    o_ref[0] = z * h + (1.0 - z) * xb


def _gru_tc(a, x, mask, wz0, wz1, wr0, wr1, wh0, wh1, bz, br, bh):
    wspec = pl.BlockSpec((_D, _D), lambda b, i: (0, 0))
    bspec = pl.BlockSpec((1, _D), lambda b, i: (0, 0))
    blk = pl.BlockSpec((1, _BN, _D), lambda b, i: (b, i, 0))
    mblk = pl.BlockSpec((1, _BN, 1), lambda b, i: (b, i, 0))
    return pl.pallas_call(
        _gru_body,
        grid=(_B, _N // _BN),
        in_specs=[blk, blk, mblk] + [wspec] * 6 + [bspec] * 3,
        out_specs=blk,
        out_shape=jax.ShapeDtypeStruct((_B, _N, _D), jnp.float32),
    )(a, x, mask, wz0, wz1, wr0, wr1, wh0, wh1, bz, br, bh)


def kernel(adj_indices, adj_values, x, mask,
           z0_weight, z0_bias, z1_weight, z1_bias,
           r0_weight, r0_bias, r1_weight, r1_bias,
           h0_weight, h0_bias, h1_weight, h1_bias):
    dst = adj_indices[:, 0, :].astype(jnp.int32)
    src = adj_indices[:, 1, :].astype(jnp.int32)
    a = _spmm_sc(x, src, dst, adj_values)
    bz = (z0_bias + z1_bias).reshape(1, _D)
    br = (r0_bias + r1_bias).reshape(1, _D)
    bh = (h0_bias + h1_bias).reshape(1, _D)
    return _gru_tc(a, x, mask, z0_weight, z1_weight, r0_weight, r1_weight,
                   h0_weight, h1_weight, bz, br, bh)


# speedup vs baseline: 4.9890x; 4.9890x over previous
import functools

import jax
import jax.numpy as jnp
from jax import lax
from jax.experimental import pallas as pl
from jax.experimental.pallas import tpu as pltpu
from jax.experimental.pallas import tpu_sc as plsc

_B, _N, _E, _D = 8, 2048, 65536, 128
_LANES = 16
_NSLICE = _D // _LANES
_NWORK = 32
_CHUNK = 2048


def _spmm_sc(x, src, dst, val):
    mesh = plsc.VectorSubcoreMesh(core_axis_name="c", subcore_axis_name="s")

    @functools.partial(
        pl.kernel,
        out_type=jax.ShapeDtypeStruct((_B, _N, _D), jnp.float32),
        mesh=mesh,
        scratch_types=[
            pltpu.VMEM((_N, _LANES), jnp.float32),
            pltpu.VMEM((_N, _LANES), jnp.float32),
            pltpu.VMEM((_CHUNK,), jnp.int32),
            pltpu.VMEM((_CHUNK,), jnp.int32),
            pltpu.VMEM((_CHUNK,), jnp.float32),
        ],
        compiler_params=pltpu.CompilerParams(
            use_tc_tiling_on_sc=False, needs_layout_passes=False),
    )
    def spmm_kernel(x_hbm, src_hbm, dst_hbm, val_hbm, a_hbm,
                    xv, acc, sb, db, vb):
        w = lax.axis_index("s") * 2 + lax.axis_index("c")

        @pl.loop(0, 2)
        def _item(k):
            itm = w + _NWORK * k
            b = itm // _NSLICE
            ds = itm % _NSLICE

            pltpu.sync_copy(x_hbm.at[b, :, pl.ds(ds * _LANES, _LANES)], xv)

            @pl.loop(0, _N)
            def _zero(i):
                acc[i, :] = jnp.zeros((_LANES,), jnp.float32)

            @pl.loop(0, _E // _CHUNK)
            def _chunk(ck):
                base = ck * _CHUNK
                pltpu.sync_copy(src_hbm.at[b, pl.ds(base, _CHUNK)], sb)
                pltpu.sync_copy(dst_hbm.at[b, pl.ds(base, _CHUNK)], db)
                pltpu.sync_copy(val_hbm.at[b, pl.ds(base, _CHUNK)], vb)

                @pl.loop(0, _CHUNK, step=_LANES)
                def _group(g):
                    sv = sb[pl.ds(g, _LANES)]
                    dv = db[pl.ds(g, _LANES)]
                    vv = vb[pl.ds(g, _LANES)]
                    for r in range(_LANES):
                        row = jnp.full((_LANES,), r, jnp.int32)
                        xr = plsc.load_gather(xv, [sv, row])
                        plsc.addupdate_scatter(acc, [dv, row], xr * vv)

            pltpu.sync_copy(acc, a_hbm.at[b, :, pl.ds(ds * _LANES, _LANES)])

    return spmm_kernel(x, src, dst, val)


_BN = 256


def _gru_body(a_ref, x_ref, m_ref, wz0, wz1, wr0, wr1, wh0, wh1,
              bz, br, bh, o_ref):
    a = a_ref[0]
    xb = x_ref[0]
    m = m_ref[0]
    dot = functools.partial(jnp.dot, preferred_element_type=jnp.float32)
    z = jax.nn.sigmoid(dot(a, wz0[...]) + dot(xb, wz1[...]) + bz[...])
    r = jax.nn.sigmoid(dot(a, wr0[...]) + dot(xb, wr1[...]) + br[...])
    h = jnp.tanh(m * (dot(a, wh0[...]) + dot(r * xb, wh1[...]) + bh[...]))
    o_ref[0] = z * h + (1.0 - z) * xb


def _gru_tc(a, x, mask, wz0, wz1, wr0, wr1, wh0, wh1, bz, br, bh):
    wspec = pl.BlockSpec((_D, _D), lambda b, i: (0, 0))
    bspec = pl.BlockSpec((1, _D), lambda b, i: (0, 0))
    blk = pl.BlockSpec((1, _BN, _D), lambda b, i: (b, i, 0))
    mblk = pl.BlockSpec((1, _BN, 1), lambda b, i: (b, i, 0))
    return pl.pallas_call(
        _gru_body,
        grid=(_B, _N // _BN),
        in_specs=[blk, blk, mblk] + [wspec] * 6 + [bspec] * 3,
        out_specs=blk,
        out_shape=jax.ShapeDtypeStruct((_B, _N, _D), jnp.float32),
    )(a, x, mask, wz0, wz1, wr0, wr1, wh0, wh1, bz, br, bh)


def kernel(adj_indices, adj_values, x, mask,
           z0_weight, z0_bias, z1_weight, z1_bias,
           r0_weight, r0_bias, r1_weight, r1_bias,
           h0_weight, h0_bias, h1_weight, h1_bias):
    dst = adj_indices[:, 0, :].astype(jnp.int32)
    src = adj_indices[:, 1, :].astype(jnp.int32)
    a = _spmm_sc(x, src, dst, adj_values)
    bz = (z0_bias + z1_bias).reshape(1, _D)
    br = (r0_bias + r1_bias).reshape(1, _D)
    bh = (h0_bias + h1_bias).reshape(1, _D)
    return _gru_tc(a, x, mask, z0_weight, z1_weight, r0_weight, r1_weight,
                   h0_weight, h1_weight, bz, br, bh)


# speedup vs baseline: 7.4279x; 1.4888x over previous
import functools

import jax
import jax.numpy as jnp
from jax import lax
from jax.experimental import pallas as pl
from jax.experimental.pallas import tpu as pltpu
from jax.experimental.pallas import tpu_sc as plsc

_B, _N, _E, _D = 8, 2048, 65536, 128
_LANES = 16
_NSLICE = _D // _LANES
_NWORK = 32
_CHUNK = 2048


def _spmm_sc(x, src, dst, val):
    mesh = plsc.VectorSubcoreMesh(core_axis_name="c", subcore_axis_name="s")

    @functools.partial(
        pl.kernel,
        out_type=jax.ShapeDtypeStruct((_B, _N, _D), jnp.float32),
        mesh=mesh,
        scratch_types=[
            pltpu.VMEM((_N, _LANES), jnp.float32),
            pltpu.VMEM((_N, _LANES), jnp.float32),
            pltpu.VMEM((_CHUNK,), jnp.int32),
            pltpu.VMEM((_CHUNK,), jnp.int32),
            pltpu.VMEM((_CHUNK,), jnp.float32),
        ],
        compiler_params=pltpu.CompilerParams(
            use_tc_tiling_on_sc=False, needs_layout_passes=False),
    )
    def spmm_kernel(x_hbm, src_hbm, dst_hbm, val_hbm, a_hbm,
                    xv, acc, sb, db, vb):
        w = lax.axis_index("s") * 2 + lax.axis_index("c")

        @pl.loop(0, 2)
        def _item(k):
            itm = w + _NWORK * k
            b = itm // _NSLICE
            ds = itm % _NSLICE

            pltpu.sync_copy(x_hbm.at[b, :, pl.ds(ds * _LANES, _LANES)], xv)

            @pl.loop(0, _N)
            def _zero(i):
                acc[i, :] = jnp.zeros((_LANES,), jnp.float32)

            @pl.loop(0, _E // _CHUNK)
            def _chunk(ck):
                base = ck * _CHUNK
                pltpu.sync_copy(src_hbm.at[b, pl.ds(base, _CHUNK)], sb)
                pltpu.sync_copy(dst_hbm.at[b, pl.ds(base, _CHUNK)], db)
                pltpu.sync_copy(val_hbm.at[b, pl.ds(base, _CHUNK)], vb)

                @plsc.parallel_loop(0, _CHUNK, step=_LANES, unroll=4)
                def _group(g):
                    sv = sb[pl.ds(g, _LANES)]
                    dv = db[pl.ds(g, _LANES)]
                    vv = vb[pl.ds(g, _LANES)]
                    for r in range(_LANES):
                        row = jnp.full((_LANES,), r, jnp.int32)
                        xr = plsc.load_gather(xv, [sv, row])
                        plsc.addupdate_scatter(acc, [dv, row], xr * vv)

            pltpu.sync_copy(acc, a_hbm.at[b, :, pl.ds(ds * _LANES, _LANES)])

    return spmm_kernel(x, src, dst, val)


_BN = 256


def _gru_body(a_ref, x_ref, m_ref, wz0, wz1, wr0, wr1, wh0, wh1,
              bz, br, bh, o_ref):
    a = a_ref[0]
    xb = x_ref[0]
    m = m_ref[0]
    dot = functools.partial(jnp.dot, preferred_element_type=jnp.float32)
    z = jax.nn.sigmoid(dot(a, wz0[...]) + dot(xb, wz1[...]) + bz[...])
    r = jax.nn.sigmoid(dot(a, wr0[...]) + dot(xb, wr1[...]) + br[...])
    h = jnp.tanh(m * (dot(a, wh0[...]) + dot(r * xb, wh1[...]) + bh[...]))
    o_ref[0] = z * h + (1.0 - z) * xb


def _gru_tc(a, x, mask, wz0, wz1, wr0, wr1, wh0, wh1, bz, br, bh):
    wspec = pl.BlockSpec((_D, _D), lambda b, i: (0, 0))
    bspec = pl.BlockSpec((1, _D), lambda b, i: (0, 0))
    blk = pl.BlockSpec((1, _BN, _D), lambda b, i: (b, i, 0))
    mblk = pl.BlockSpec((1, _BN, 1), lambda b, i: (b, i, 0))
    return pl.pallas_call(
        _gru_body,
        grid=(_B, _N // _BN),
        in_specs=[blk, blk, mblk] + [wspec] * 6 + [bspec] * 3,
        out_specs=blk,
        out_shape=jax.ShapeDtypeStruct((_B, _N, _D), jnp.float32),
    )(a, x, mask, wz0, wz1, wr0, wr1, wh0, wh1, bz, br, bh)


def kernel(adj_indices, adj_values, x, mask,
           z0_weight, z0_bias, z1_weight, z1_bias,
           r0_weight, r0_bias, r1_weight, r1_bias,
           h0_weight, h0_bias, h1_weight, h1_bias):
    dst = adj_indices[:, 0, :].astype(jnp.int32)
    src = adj_indices[:, 1, :].astype(jnp.int32)
    a = _spmm_sc(x, src, dst, adj_values)
    bz = (z0_bias + z1_bias).reshape(1, _D)
    br = (r0_bias + r1_bias).reshape(1, _D)
    bh = (h0_bias + h1_bias).reshape(1, _D)
    return _gru_tc(a, x, mask, z0_weight, z1_weight, r0_weight, r1_weight,
                   h0_weight, h1_weight, bz, br, bh)
